# baseline (speedup 1.0000x reference)
import functools

import jax
import jax.numpy as jnp
from jax import lax
from jax.experimental import pallas as pl
from jax.experimental.pallas import tpu as pltpu
from jax.experimental.pallas import tpu_sc as plsc

_NTOKENS = 100000
_NINP = 128
_P_E = 0.1
_P_I = 0.65

_B = 1024
_L = 200

_NC = 2
_NS = 16
_NW = _NC * _NS

_CHUNK = 64
_CPB = _B // _CHUNK
_NCH = _L * _CPB
_CPW = _NCH // _NW
_NBUF = 4
_MIPAD = 208


def _sc_body(xt_hbm, maske_hbm, weight_hbm, mi_hbm, out_hbm,
             idx_v, mval_v, rows_v, out_v, oidx_v, mi_v,
             sem_mval, sem_row0, sem_row1, sem_row2, sem_row3,
             sem_out0, sem_out1, sem_out2, sem_out3):
    wid = lax.axis_index("s") * _NC + lax.axis_index("c")
    sem_row = (sem_row0, sem_row1, sem_row2, sem_row3)
    sem_out = (sem_out0, sem_out1, sem_out2, sem_out3)

    pltpu.sync_copy(xt_hbm.at[wid], idx_v)

    for nb in range(_NBUF):
        pltpu.async_copy(weight_hbm.at[idx_v.at[nb]], rows_v.at[nb],
                         sem_row[nb])

    for j in range(_CPW):
        pltpu.async_copy(maske_hbm.at[idx_v.at[j]], mval_v.at[j], sem_mval)

    lbase = pl.multiple_of(((wid * _CPW) >> 4) & ~7, 8)
    pltpu.sync_copy(mi_hbm.at[pl.ds(lbase, 16)], mi_v)

    for j in range(_CPW):
        pltpu.make_async_copy(maske_hbm.at[idx_v.at[0]], mval_v.at[0],
                              sem_mval).wait()

    def iter_body(j2, carry):
        for nb in range(_NBUF):
            c = _NBUF * j2 + nb
            fr = wid * _CPW + c
            l = fr >> 4
            b0 = (fr & 15) * _CHUNK

            pltpu.make_async_copy(weight_hbm.at[idx_v.at[0]], rows_v.at[nb],
                                  sem_row[nb]).wait()

            @pl.when(j2 > 0)
            def _():
                pltpu.make_async_copy(out_v.at[nb], out_hbm.at[oidx_v.at[nb]],
                                      sem_out[nb]).wait()

            mrow = [mi_v[l - lbase, pl.ds(16 * d, 16)] for d in range(8)]

            def grp_body(g, acc):
                sv = mval_v[c, pl.ds(16 * g, 16)]
                for r16 in range(16):
                    r = 16 * g + r16
                    s = sv[r16]
                    for d in range(8):
                        sl = pl.ds(16 * d, 16)
                        out_v[nb, r, sl] = rows_v[nb, r, sl] * s * mrow[d]
                return acc

            lax.fori_loop(0, _CHUNK // 16, grp_body, 0)

            @pl.when(j2 < _CPW // _NBUF - 1)
            def _():
                pltpu.async_copy(weight_hbm.at[idx_v.at[c + _NBUF]],
                                 rows_v.at[nb], sem_row[nb])

            lane = lax.iota(jnp.int32, 16)
            for k in range(_CHUNK // 16):
                oidx_v[nb, pl.ds(16 * k, 16)] = (b0 + 16 * k + lane) * _L + l
            pltpu.async_copy(out_v.at[nb], out_hbm.at[oidx_v.at[nb]],
                             sem_out[nb])
        return carry

    lax.fori_loop(0, _CPW // _NBUF, iter_body, 0)

    for nb in range(_NBUF):
        pltpu.make_async_copy(out_v.at[nb], out_hbm.at[oidx_v.at[nb]],
                              sem_out[nb]).wait()


_launch = functools.partial(
    pl.kernel,
    mesh=plsc.VectorSubcoreMesh(core_axis_name="c", subcore_axis_name="s"),
    out_type=jax.ShapeDtypeStruct((_B * _L, _NINP), jnp.float32),
    scratch_types=[
        pltpu.VMEM((_CPW, _CHUNK), jnp.int32),
        pltpu.VMEM((_CPW, _CHUNK), jnp.float32),
        pltpu.VMEM((_NBUF, _CHUNK, _NINP), jnp.float32),
        pltpu.VMEM((_NBUF, _CHUNK, _NINP), jnp.float32),
        pltpu.VMEM((_NBUF, _CHUNK), jnp.int32),
        pltpu.VMEM((16, _NINP), jnp.float32),
        pltpu.SemaphoreType.DMA,
        pltpu.SemaphoreType.DMA,
        pltpu.SemaphoreType.DMA,
        pltpu.SemaphoreType.DMA,
        pltpu.SemaphoreType.DMA,
        pltpu.SemaphoreType.DMA,
        pltpu.SemaphoreType.DMA,
        pltpu.SemaphoreType.DMA,
        pltpu.SemaphoreType.DMA,
    ],
)(_sc_body)


@jax.jit
def kernel(X, weight):
    mask_e = jax.random.bernoulli(
        jax.random.key(1), 1.0 - _P_E, (_NTOKENS, 1)).astype(weight.dtype)
    mask_e = (mask_e / (1.0 - _P_E))[:, 0]
    mask_i = jax.random.bernoulli(
        jax.random.key(2), 1.0 - _P_I, (1, _L, _NINP)).astype(weight.dtype)
    mask_i = (mask_i / (1.0 - _P_I))[0]
    mask_i = jnp.pad(mask_i, ((0, _MIPAD - _L), (0, 0)))
    xt = X.T.reshape(_NW, _CPW, _CHUNK)
    out2 = _launch(xt, mask_e, weight, mask_i)
    return out2.reshape(_B, _L, _NINP)
